# baseline (speedup 1.0000x reference)
import functools

import jax
import jax.numpy as jnp
from jax import lax
from jax.experimental import pallas as pl
from jax.experimental.pallas import tpu as pltpu
from jax.experimental.pallas import tpu_sc as plsc

N = 10000
E = 320000
D = 8
NC, NS = 2, 16
NW = NC * NS
CH = 128
ER = E // CH
MAXC = 79
NPAD = 10240
STRIPE = NPAD // NS
PR = N * D // 128
PRP = NPAD * D // 128


def _sc_mesh():
    return plsc.VectorSubcoreMesh(
        core_axis_name="c", subcore_axis_name="s", num_cores=NC, num_subcores=NS
    )


def _tile_range(wid):
    r0 = (wid * ER) // NW
    r1 = ((wid + 1) * ER) // NW
    return r0, r1 - r0


def _edges_body(e_ref, s_ref, d_ref):
    v = e_ref[...]
    s_ref[...] = v[0].reshape(ER, CH)
    d_ref[...] = v[1].reshape(ER, CH)


def _edges_call(e):
    return pl.pallas_call(
        _edges_body,
        out_shape=(
            jax.ShapeDtypeStruct((ER, CH), jnp.int32),
            jax.ShapeDtypeStruct((ER, CH), jnp.int32),
        ),
    )(e)


@functools.partial(
    pl.kernel,
    out_type=jax.ShapeDtypeStruct((NC, NPAD, D), jnp.float32),
    mesh=_sc_mesh(),
    compiler_params=pltpu.CompilerParams(use_tc_tiling_on_sc=False),
    scratch_types=[
        pltpu.VMEM((MAXC, CH), jnp.int32),
        pltpu.VMEM((CH, D), jnp.float32),
        pltpu.VMEM_SHARED((NPAD, D), jnp.float32),
        pltpu.SemaphoreType.DMA,
    ],
)
def _deg_kernel(dst_hbm, zeros_hbm, ones_hbm, out_hbm, idx_v, ones_v, deg_sh, dsem):
    cid = lax.axis_index("c")
    sid = lax.axis_index("s")
    wid = cid * NS + sid
    r0, cnt = _tile_range(wid)
    pltpu.sync_copy(
        zeros_hbm.at[pl.ds(sid * STRIPE, STRIPE)],
        deg_sh.at[pl.ds(sid * STRIPE, STRIPE)],
    )
    pltpu.sync_copy(dst_hbm.at[pl.ds(r0, MAXC)], idx_v)
    pltpu.sync_copy(ones_hbm, ones_v)
    plsc.subcore_barrier()

    def body(j, carry):
        @pl.when(j >= 8)
        def _():
            pltpu.make_async_copy(ones_v, deg_sh.at[idx_v.at[j - 8]], dsem).wait()

        pltpu.async_copy(ones_v, deg_sh.at[idx_v.at[j]], dsem, add=True)
        return carry

    lax.fori_loop(0, cnt, body, 0)

    def drain(j, carry):
        pltpu.make_async_copy(ones_v, deg_sh.at[idx_v.at[j]], dsem).wait()
        return carry

    lax.fori_loop(cnt - 8, cnt, drain, 0)
    plsc.subcore_barrier()
    pltpu.sync_copy(
        deg_sh.at[pl.ds(sid * STRIPE, STRIPE)],
        out_hbm.at[cid, pl.ds(sid * STRIPE, STRIPE)],
    )


@functools.partial(
    pl.kernel,
    out_type=jax.ShapeDtypeStruct((NC, NPAD, D), jnp.float32),
    mesh=_sc_mesh(),
    compiler_params=pltpu.CompilerParams(use_tc_tiling_on_sc=False),
    scratch_types=(
        [
            pltpu.VMEM((MAXC, CH), jnp.int32),
            pltpu.VMEM((MAXC, CH), jnp.int32),
            pltpu.VMEM_SHARED((NPAD, D), jnp.float32),
        ]
        + [pltpu.VMEM((CH, D), jnp.float32)] * 8
        + [pltpu.SemaphoreType.DMA] * 16
    ),
)
def _msg_kernel(y_hbm, src_hbm, dst_hbm, zeros_hbm, out_hbm,
                src_v, dst_v, acc_sh, *bufs_and_sems):
    msg = bufs_and_sems[0:8]
    gsem = bufs_and_sems[8:16]
    ssem = bufs_and_sems[16:24]
    cid = lax.axis_index("c")
    sid = lax.axis_index("s")
    wid = cid * NS + sid
    r0, cnt = _tile_range(wid)
    pltpu.sync_copy(
        zeros_hbm.at[pl.ds(sid * STRIPE, STRIPE)],
        acc_sh.at[pl.ds(sid * STRIPE, STRIPE)],
    )
    pltpu.sync_copy(src_hbm.at[pl.ds(r0, MAXC)], src_v)
    pltpu.sync_copy(dst_hbm.at[pl.ds(r0, MAXC)], dst_v)
    plsc.subcore_barrier()

    def step(j, b):
        pb = (b + 4) % 8

        @pl.when(j >= 4)
        def _():
            pltpu.make_async_copy(
                msg[pb], acc_sh.at[dst_v.at[j - 4]], ssem[pb]).wait()

        @pl.when(j + 4 < cnt)
        def _():
            pltpu.async_copy(y_hbm.at[src_v.at[j + 4]], msg[pb], gsem[pb])

        pltpu.make_async_copy(y_hbm.at[src_v.at[j]], msg[b], gsem[b]).wait()
        pltpu.async_copy(msg[b], acc_sh.at[dst_v.at[j]], ssem[b], add=True)

    for b in range(4):
        pltpu.async_copy(y_hbm.at[src_v.at[b]], msg[b], gsem[b])

    grp = cnt // 8

    def body(i, carry):
        for b in range(8):
            step(8 * i + b, b)
        return carry

    lax.fori_loop(0, grp, body, 0)

    for b in range(8):
        j = 8 * grp + b

        @pl.when(j < cnt)
        def _():
            step(j, b)

    for b in range(8):
        jb = cnt - 1 - ((cnt - 1 - b) % 8)

        @pl.when(jb >= cnt - 4)
        def _():
            pltpu.make_async_copy(msg[b], acc_sh.at[dst_v.at[jb]], ssem[b]).wait()

    plsc.subcore_barrier()
    pltpu.sync_copy(
        acc_sh.at[pl.ds(sid * STRIPE, STRIPE)],
        out_hbm.at[cid, pl.ds(sid * STRIPE, STRIPE)],
    )


def _mm_body(x16_ref, wd_ref, xwv_ref):
    xwv_ref[...] = jnp.dot(
        x16_ref[...], wd_ref[...], preferred_element_type=jnp.float32)


def _mm_call(x16, wd):
    return pl.pallas_call(
        _mm_body,
        out_shape=jax.ShapeDtypeStruct((PR, 128), jnp.float32),
    )(x16, wd)


def _scale_body(xwv_ref, degv_ref, yv_ref):
    deg = 1.0 + degv_ref[0, :PR, :] + degv_ref[1, :PR, :]
    yv_ref[pl.ds(0, PR), :] = xwv_ref[...] * lax.rsqrt(deg)
    yv_ref[pl.ds(PR, PRP - PR), :] = jnp.zeros((PRP - PR, 128), jnp.float32)


def _scale_call(xwv, degv):
    return pl.pallas_call(
        _scale_body,
        out_shape=jax.ShapeDtypeStruct((PRP, 128), jnp.float32),
    )(xwv, degv)


def _head_body(yv_ref, accv_ref, degv_ref, bgv_ref, mz_ref, bov_ref, o_ref):
    deg = 1.0 + degv_ref[0, :PR, :] + degv_ref[1, :PR, :]
    dis = lax.rsqrt(deg)
    s = (accv_ref[0, :PR, :] + accv_ref[1, :PR, :] + yv_ref[:PR, :]) * dis
    h8 = jnp.maximum(s + bgv_ref[...], 0.0)
    o_ref[...] = (
        h8
        + jnp.dot(h8, mz_ref[...], preferred_element_type=jnp.float32)
        + bov_ref[...]
    )


def _head_call(yv, accv, degv, bgv, mz, bov):
    return pl.pallas_call(
        _head_body,
        out_shape=jax.ShapeDtypeStruct((PR, 128), jnp.float32),
    )(yv, accv, degv, bgv, mz, bov)


def kernel(x, edge_index, W_gcn, b_gcn, W_out, b_out):
    src2, dst2 = _edges_call(edge_index.astype(jnp.int32))
    zeros2 = jnp.zeros((NPAD, D), jnp.float32)
    ones8 = jnp.ones((CH, D), jnp.float32)

    degp = _deg_kernel(dst2, zeros2, ones8)
    degv = degp.reshape(NC, PRP, 128)

    eye16 = jnp.eye(16, dtype=jnp.float32)
    wd = jnp.kron(eye16, jnp.pad(W_gcn, ((0, 0), (0, D - 3))))
    xwv = _mm_call(x.reshape(PR, 16 * 128), wd)
    yv = _scale_call(xwv, degv)

    accp = _msg_kernel(yv.reshape(NPAD, D), src2, dst2, zeros2)
    accv = accp.reshape(NC, PRP, 128)

    bgv = jnp.tile(jnp.pad(b_gcn, (0, D - 3)), 16).reshape(1, 128)
    mz = jnp.kron(eye16, jnp.pad(W_out, ((0, D - 3), (3, 1))))
    bov = jnp.tile(jnp.pad(b_out, (3, 1)), 16).reshape(1, 128)
    ov = _head_call(yv, accv, degv, bgv, mz, bov)

    o = ov.reshape(N, D)
    h = o[:, :3]
    z = o[:, 3:7]
    return (h, z)


# speedup vs baseline: 1.0189x; 1.0189x over previous
import functools

import jax
import jax.numpy as jnp
from jax import lax
from jax.experimental import pallas as pl
from jax.experimental.pallas import tpu as pltpu
from jax.experimental.pallas import tpu_sc as plsc

N = 10000
E = 320000
D = 8
NC, NS = 2, 16
NW = NC * NS
CH = 128
ER = E // CH
MAXC = 79
NPAD = 10240
STRIPE = NPAD // NS
PR = N * D // 128
PRP = NPAD * D // 128


def _sc_mesh():
    return plsc.VectorSubcoreMesh(
        core_axis_name="c", subcore_axis_name="s", num_cores=NC, num_subcores=NS
    )


def _tile_range(wid):
    r0 = (wid * ER) // NW
    r1 = ((wid + 1) * ER) // NW
    return r0, r1 - r0


def _edges_body(e_ref, s_ref, d_ref):
    v = e_ref[...]
    s_ref[...] = v[0].reshape(ER, CH)
    d_ref[...] = v[1].reshape(ER, CH)


def _edges_call(e):
    return pl.pallas_call(
        _edges_body,
        out_shape=(
            jax.ShapeDtypeStruct((ER, CH), jnp.int32),
            jax.ShapeDtypeStruct((ER, CH), jnp.int32),
        ),
    )(e)


@functools.partial(
    pl.kernel,
    out_type=jax.ShapeDtypeStruct((NC, NPAD, D), jnp.float32),
    mesh=_sc_mesh(),
    compiler_params=pltpu.CompilerParams(use_tc_tiling_on_sc=False),
    scratch_types=[
        pltpu.VMEM((MAXC, CH), jnp.int32),
        pltpu.VMEM((CH, D), jnp.float32),
        pltpu.VMEM_SHARED((NPAD, D), jnp.float32),
        pltpu.SemaphoreType.DMA,
    ],
)
def _deg_kernel(dst_hbm, zeros_hbm, ones_hbm, out_hbm, idx_v, ones_v, deg_sh, dsem):
    cid = lax.axis_index("c")
    sid = lax.axis_index("s")
    wid = cid * NS + sid
    r0, cnt = _tile_range(wid)
    pltpu.sync_copy(
        zeros_hbm.at[pl.ds(sid * STRIPE, STRIPE)],
        deg_sh.at[pl.ds(sid * STRIPE, STRIPE)],
    )
    pltpu.sync_copy(dst_hbm.at[pl.ds(r0, MAXC)], idx_v)
    pltpu.sync_copy(ones_hbm, ones_v)
    plsc.subcore_barrier()

    def body(j, carry):
        @pl.when(j >= 8)
        def _():
            pltpu.make_async_copy(ones_v, deg_sh.at[idx_v.at[j - 8]], dsem).wait()

        pltpu.async_copy(ones_v, deg_sh.at[idx_v.at[j]], dsem, add=True)
        return carry

    lax.fori_loop(0, cnt, body, 0)

    def drain(j, carry):
        pltpu.make_async_copy(ones_v, deg_sh.at[idx_v.at[j]], dsem).wait()
        return carry

    lax.fori_loop(cnt - 8, cnt, drain, 0)
    plsc.subcore_barrier()
    pltpu.sync_copy(
        deg_sh.at[pl.ds(sid * STRIPE, STRIPE)],
        out_hbm.at[cid, pl.ds(sid * STRIPE, STRIPE)],
    )


@functools.partial(
    pl.kernel,
    out_type=jax.ShapeDtypeStruct((NC, NPAD, D), jnp.float32),
    mesh=_sc_mesh(),
    compiler_params=pltpu.CompilerParams(use_tc_tiling_on_sc=False),
    scratch_types=(
        [
            pltpu.VMEM((MAXC, CH), jnp.int32),
            pltpu.VMEM((MAXC, CH), jnp.int32),
            pltpu.VMEM_SHARED((NPAD, D), jnp.float32),
        ]
        + [pltpu.VMEM((CH, D), jnp.float32)] * 8
        + [pltpu.SemaphoreType.DMA] * 16
    ),
)
def _msg_kernel(y_hbm, src_hbm, dst_hbm, zeros_hbm, out_hbm,
                src_v, dst_v, acc_sh, *bufs_and_sems):
    msg = bufs_and_sems[0:8]
    gsem = bufs_and_sems[8:16]
    ssem = bufs_and_sems[16:24]
    cid = lax.axis_index("c")
    sid = lax.axis_index("s")
    wid = cid * NS + sid
    r0, cnt = _tile_range(wid)
    pltpu.sync_copy(
        zeros_hbm.at[pl.ds(sid * STRIPE, STRIPE)],
        acc_sh.at[pl.ds(sid * STRIPE, STRIPE)],
    )
    pltpu.sync_copy(src_hbm.at[pl.ds(r0, MAXC)], src_v)
    pltpu.sync_copy(dst_hbm.at[pl.ds(r0, MAXC)], dst_v)
    plsc.subcore_barrier()

    def step(j, b):
        pb = (b + 4) % 8

        @pl.when(j >= 4)
        def _():
            pltpu.make_async_copy(
                msg[pb], acc_sh.at[dst_v.at[j - 4]], ssem[pb]).wait()

        @pl.when(j + 4 < cnt)
        def _():
            pltpu.async_copy(y_hbm.at[src_v.at[j + 4]], msg[pb], gsem[pb])

        pltpu.make_async_copy(y_hbm.at[src_v.at[j]], msg[b], gsem[b]).wait()
        pltpu.async_copy(msg[b], acc_sh.at[dst_v.at[j]], ssem[b], add=True)

    for b in range(4):
        pltpu.async_copy(y_hbm.at[src_v.at[b]], msg[b], gsem[b])

    grp = cnt // 8

    def body(i, carry):
        for b in range(8):
            step(8 * i + b, b)
        return carry

    lax.fori_loop(0, grp, body, 0)

    for b in range(8):
        j = 8 * grp + b

        @pl.when(j < cnt)
        def _():
            step(j, b)

    for b in range(8):
        jb = cnt - 1 - ((cnt - 1 - b) % 8)

        @pl.when(jb >= cnt - 4)
        def _():
            pltpu.make_async_copy(msg[b], acc_sh.at[dst_v.at[jb]], ssem[b]).wait()

    plsc.subcore_barrier()
    pltpu.sync_copy(
        acc_sh.at[pl.ds(sid * STRIPE, STRIPE)],
        out_hbm.at[cid, pl.ds(sid * STRIPE, STRIPE)],
    )


def _mm_body(x16_ref, wd_ref, xwv_ref):
    xwv_ref[...] = jnp.dot(
        x16_ref[...], wd_ref[...], preferred_element_type=jnp.float32)


def _mm_call(x16, wd):
    return pl.pallas_call(
        _mm_body,
        out_shape=jax.ShapeDtypeStruct((PR, 128), jnp.float32),
    )(x16, wd)


def _scale_body(xwv_ref, degv_ref, yv_ref):
    deg = 1.0 + degv_ref[0, :PR, :] + degv_ref[1, :PR, :]
    yv_ref[pl.ds(0, PR), :] = xwv_ref[...] * lax.rsqrt(deg)
    yv_ref[pl.ds(PR, PRP - PR), :] = jnp.zeros((PRP - PR, 128), jnp.float32)


def _scale_call(xwv, degv):
    return pl.pallas_call(
        _scale_body,
        out_shape=jax.ShapeDtypeStruct((PRP, 128), jnp.float32),
    )(xwv, degv)


def _head_body(yv_ref, accv_ref, degv_ref, bgv_ref, mz_ref, bov_ref,
               h_ref, z_ref):
    deg = 1.0 + degv_ref[0, :PR, :] + degv_ref[1, :PR, :]
    dis = lax.rsqrt(deg)
    s = (accv_ref[0, :PR, :] + accv_ref[1, :PR, :] + yv_ref[:PR, :]) * dis
    h8 = jnp.maximum(s + bgv_ref[...], 0.0)
    h_ref[...] = h8
    z_ref[...] = (
        jnp.dot(h8, mz_ref[...], preferred_element_type=jnp.float32)
        + bov_ref[...]
    )


def _head_call(yv, accv, degv, bgv, mz, bov):
    return pl.pallas_call(
        _head_body,
        out_shape=(
            jax.ShapeDtypeStruct((PR, 128), jnp.float32),
            jax.ShapeDtypeStruct((PR, 128), jnp.float32),
        ),
    )(yv, accv, degv, bgv, mz, bov)


def kernel(x, edge_index, W_gcn, b_gcn, W_out, b_out):
    src2, dst2 = _edges_call(edge_index.astype(jnp.int32))
    zeros2 = jnp.zeros((NPAD, D), jnp.float32)
    ones8 = jnp.ones((CH, D), jnp.float32)

    degp = _deg_kernel(dst2, zeros2, ones8)
    degv = degp.reshape(NC, PRP, 128)

    eye16 = jnp.eye(16, dtype=jnp.float32)
    wd = jnp.kron(eye16, jnp.pad(W_gcn, ((0, 0), (0, D - 3))))
    xwv = _mm_call(x.reshape(PR, 16 * 128), wd)
    yv = _scale_call(xwv, degv)

    accp = _msg_kernel(yv.reshape(NPAD, D), src2, dst2, zeros2)
    accv = accp.reshape(NC, PRP, 128)

    bgv = jnp.tile(jnp.pad(b_gcn, (0, D - 3)), 16).reshape(1, 128)
    mz = jnp.kron(eye16, jnp.pad(W_out, ((0, D - 3), (0, D - 4))))
    bov = jnp.tile(jnp.pad(b_out, (0, D - 4)), 16).reshape(1, 128)
    hv, zv = _head_call(yv, accv, degv, bgv, mz, bov)

    h = hv.reshape(N, D)[:, :3]
    z = zv.reshape(N, D)[:, :4]
    return (h, z)
